# Initial kernel scaffold; baseline (speedup 1.0000x reference)
import jax
import jax.numpy as jnp
from jax.experimental import pallas as pl


def kernel(x, pe):
    raise NotImplementedError("write your pallas kernel here")


# speedup vs baseline: 2.8539x; 2.8539x over previous
import jax
import jax.numpy as jnp
from jax.experimental import pallas as pl


def _add_kernel(x_ref, pe_ref, out_ref):
    out_ref[...] = x_ref[...] + pe_ref[...]


def kernel(x, pe):
    batch, seq_len, d_model = x.shape
    s_blk = 512
    grid = (seq_len // s_blk, batch)
    return pl.pallas_call(
        _add_kernel,
        grid=grid,
        in_specs=[
            pl.BlockSpec((1, s_blk, d_model), lambda i, b: (b, i, 0)),
            pl.BlockSpec((s_blk, d_model), lambda i, b: (i, 0)),
        ],
        out_specs=pl.BlockSpec((1, s_blk, d_model), lambda i, b: (b, i, 0)),
        out_shape=jax.ShapeDtypeStruct(x.shape, x.dtype),
    )(x, pe[:seq_len])


# speedup vs baseline: 3.1764x; 1.1130x over previous
import jax
import jax.numpy as jnp
from jax import lax
from jax.experimental import pallas as pl
from jax.experimental.pallas import tpu as pltpu
from jax.experimental.pallas import tpu_sc as plsc

_NW = 32
_CH_ROWS = 32


def _sc_body(x_hbm, pe_hbm, out_hbm, pe_buf, x_buf, sem):
    del sem
    wid = lax.axis_index("s") * 2 + lax.axis_index("c")
    total_rows = x_hbm.shape[0]
    pe_rows = pe_hbm.shape[0]
    d = pe_hbm.shape[1]
    batch = total_rows // pe_rows
    w_rows = pe_rows // _NW
    n_chunks = w_rows // _CH_ROWS
    n_vec = _CH_ROWS * d // 16
    vec_per_row = d // 16

    def chunk_body(c, _):
        s0 = wid * w_rows + c * _CH_ROWS
        pltpu.sync_copy(pe_hbm.at[pl.ds(s0, _CH_ROWS)], pe_buf)

        def batch_body(b, _):
            r0 = b * pe_rows + s0
            pltpu.sync_copy(x_hbm.at[pl.ds(r0, _CH_ROWS)], x_buf)

            def add_body(i, _):
                r = lax.div(i, vec_per_row)
                col = lax.rem(i, vec_per_row) * 16
                plsc.addupdate(
                    x_buf.at[r, pl.ds(col, 16)], pe_buf[r, pl.ds(col, 16)]
                )
                return 0

            lax.fori_loop(0, n_vec, add_body, 0)
            pltpu.sync_copy(x_buf, out_hbm.at[pl.ds(r0, _CH_ROWS)])
            return 0

        lax.fori_loop(0, batch, batch_body, 0)
        return 0

    lax.fori_loop(0, n_chunks, chunk_body, 0)


def kernel(x, pe):
    batch, seq_len, d_model = x.shape
    xr = x.reshape(batch * seq_len, d_model)
    mesh = plsc.VectorSubcoreMesh(core_axis_name="c", subcore_axis_name="s")
    run = pl.kernel(
        _sc_body,
        out_type=jax.ShapeDtypeStruct((batch * seq_len, d_model), x.dtype),
        mesh=mesh,
        scratch_types=[
            pltpu.VMEM((_CH_ROWS, d_model), jnp.float32),
            pltpu.VMEM((_CH_ROWS, d_model), jnp.float32),
            pltpu.SemaphoreType.DMA,
        ],
    )
    return run(xr, pe[:seq_len]).reshape(batch, seq_len, d_model)


# speedup vs baseline: 3.2870x; 1.0348x over previous
import jax
import jax.numpy as jnp
from jax.experimental import pallas as pl


def _add_kernel(x_ref, pe_ref, out_ref):
    out_ref[...] = x_ref[...] + pe_ref[...]


def kernel(x, pe):
    batch, seq_len, d_model = x.shape
    s_blk = 4096
    d_blk = 512
    grid = (seq_len // s_blk, d_model // d_blk, batch)
    return pl.pallas_call(
        _add_kernel,
        grid=grid,
        in_specs=[
            pl.BlockSpec((1, s_blk, d_blk), lambda i, j, b: (b, i, j)),
            pl.BlockSpec((s_blk, d_blk), lambda i, j, b: (i, j)),
        ],
        out_specs=pl.BlockSpec((1, s_blk, d_blk), lambda i, j, b: (b, i, j)),
        out_shape=jax.ShapeDtypeStruct(x.shape, x.dtype),
    )(x, pe[:seq_len])


# speedup vs baseline: 3.3108x; 1.0072x over previous
import jax
import jax.numpy as jnp
from jax.experimental import pallas as pl


def _add_kernel(x_ref, pe_ref, out_ref):
    out_ref[...] = x_ref[...] + pe_ref[...]


def kernel(x, pe):
    batch, seq_len, d_model = x.shape
    s_blk = 512
    grid = (seq_len // s_blk,)
    return pl.pallas_call(
        _add_kernel,
        grid=grid,
        in_specs=[
            pl.BlockSpec((batch, s_blk, d_model), lambda i: (0, i, 0)),
            pl.BlockSpec((s_blk, d_model), lambda i: (i, 0)),
        ],
        out_specs=pl.BlockSpec((batch, s_blk, d_model), lambda i: (0, i, 0)),
        out_shape=jax.ShapeDtypeStruct(x.shape, x.dtype),
    )(x, pe[:seq_len])
